# baseline (speedup 1.0000x reference)
import functools

import jax
import jax.numpy as jnp
from jax import lax
from jax.experimental import pallas as pl
from jax.experimental.pallas import tpu as pltpu
from jax.experimental.pallas import tpu_sc as plsc

_VOCAB = 1000000
_D = 64
_B = 16384
_L = 200
_N = _B * _L

_VLB = 32768
_NVB = -(-_VOCAB // _VLB)

_NC = 2
_NS = 16
_NW = _NC * _NS
_PER_W = _N // _NW
_CH = 12800
_NCHUNK = _PER_W // _CH

_PL = 100
_NPB = _L // _PL


def _project_body(embt_ref, w_ref, out_ref):
    xt = embt_ref[...]
    w = w_ref[...]
    y = lax.dot_general(w, xt, (((1,), (0,)), ((), ())),
                        preferred_element_type=jnp.float32)
    out_ref[...] = y[0]


def _project_table(embt, fc_w):
    return pl.pallas_call(
        _project_body,
        grid=(_NVB,),
        in_specs=[
            pl.BlockSpec((_D, _VLB), lambda i: (0, i)),
            pl.BlockSpec((1, _D), lambda i: (0, 0)),
        ],
        out_specs=pl.BlockSpec((_VLB,), lambda i: (i,)),
        out_shape=jax.ShapeDtypeStruct((_VOCAB,), jnp.float32),
    )(embt, fc_w)


def _sc_gather(p, idx_flat):
    mesh = plsc.VectorSubcoreMesh(core_axis_name="c", subcore_axis_name="s")

    @functools.partial(
        pl.kernel,
        out_type=jax.ShapeDtypeStruct((_N,), jnp.float32),
        mesh=mesh,
        scratch_types=[
            pltpu.VMEM((_CH,), jnp.int32),
            pltpu.VMEM((_CH,), jnp.int32),
            pltpu.VMEM((_CH,), jnp.int32),
            pltpu.VMEM((_CH,), jnp.float32),
            pltpu.VMEM((_CH,), jnp.float32),
            pltpu.SemaphoreType.DMA,
            pltpu.SemaphoreType.DMA,
            pltpu.SemaphoreType.DMA,
            pltpu.SemaphoreType.DMA,
            pltpu.SemaphoreType.DMA,
        ],
    )
    def gather_kernel(p_hbm, idx_hbm, out_hbm, idx0, idx1, idx2,
                      val0, val1, sem_i, sem_g0, sem_g1, sem_o0, sem_o1):
        wid = lax.axis_index("s") * _NC + lax.axis_index("c")
        base = wid * _PER_W
        ibufs = (idx0, idx1, idx2)
        vbufs = (val0, val1)
        gsems = (sem_g0, sem_g1)
        osems = (sem_o0, sem_o1)

        h_idx = [None] * _NCHUNK
        h_g = [None] * _NCHUNK
        h_out = [None] * _NCHUNK
        for k in range(min(2, _NCHUNK)):
            h_idx[k] = pltpu.async_copy(
                idx_hbm.at[pl.ds(base + k * _CH, _CH)], ibufs[k % 3], sem_i)
        for k in range(_NCHUNK):
            h_idx[k].wait()
            if k >= 2:
                h_out[k - 2].wait()
            h_g[k] = pltpu.async_copy(p_hbm.at[ibufs[k % 3]],
                                      vbufs[k % 2], gsems[k % 2])
            if k == 0 and _NCHUNK > 2:
                h_idx[2] = pltpu.async_copy(
                    idx_hbm.at[pl.ds(base + 2 * _CH, _CH)], ibufs[2], sem_i)
            if k >= 1:
                h_g[k - 1].wait()
                h_out[k - 1] = pltpu.async_copy(
                    vbufs[(k - 1) % 2],
                    out_hbm.at[pl.ds(base + (k - 1) * _CH, _CH)],
                    osems[(k - 1) % 2])
            if k >= 1 and k + 2 < _NCHUNK:
                off = base + (k + 2) * _CH
                h_idx[k + 2] = pltpu.async_copy(
                    idx_hbm.at[pl.ds(off, _CH)], ibufs[(k + 2) % 3], sem_i)
        h_g[_NCHUNK - 1].wait()
        h_out[_NCHUNK - 1] = pltpu.async_copy(
            vbufs[(_NCHUNK - 1) % 2],
            out_hbm.at[pl.ds(base + (_NCHUNK - 1) * _CH, _CH)],
            osems[(_NCHUNK - 1) % 2])
        for k in range(max(0, _NCHUNK - 2), _NCHUNK):
            h_out[k].wait()

    return gather_kernel(p, idx_flat)


def _pool_body(vals_ref, mask_ref, b_ref, out_ref, acc_s, acc_m):
    i = pl.program_id(0)

    @pl.when(i == 0)
    def _init():
        acc_s[...] = jnp.zeros((_B,), jnp.float32)
        acc_m[...] = jnp.zeros((_B,), jnp.float32)

    v = vals_ref[...]
    m = mask_ref[...].astype(jnp.float32)
    s_part = acc_s[...]
    m_part = acc_m[...]
    for s in range(_PL):
        sl = slice(s * _B, (s + 1) * _B)
        s_part = s_part + v[sl] * m[sl]
        m_part = m_part + m[sl]
    acc_s[...] = s_part
    acc_m[...] = m_part

    @pl.when(i == _NPB - 1)
    def _fin():
        out_ref[...] = s_part / m_part + b_ref[0, 0]


def _pool(vals, mask_flat, fc_b):
    b2d = fc_b.reshape(1, 1)
    return pl.pallas_call(
        _pool_body,
        grid=(_NPB,),
        in_specs=[
            pl.BlockSpec((_PL * _B,), lambda i: (i,)),
            pl.BlockSpec((_PL * _B,), lambda i: (i,)),
            pl.BlockSpec((1, 1), lambda i: (0, 0)),
        ],
        out_specs=pl.BlockSpec((_B,), lambda i: (0,)),
        out_shape=jax.ShapeDtypeStruct((_B,), jnp.float32),
        scratch_shapes=[
            pltpu.VMEM((_B,), jnp.float32),
            pltpu.VMEM((_B,), jnp.float32),
        ],
    )(vals, mask_flat, b2d)


def kernel(input_ids, attention_mask, emb_table, fc_w, fc_b):
    p = _project_table(emb_table.T, fc_w)
    idx_flat = input_ids.astype(jnp.int32).T.reshape(_N)
    mask_flat = attention_mask.astype(jnp.int32).T.reshape(_N)
    vals = _sc_gather(p, idx_flat)
    logits = _pool(vals, mask_flat, fc_b)
    return logits.reshape(_B, 1)


# speedup vs baseline: 1.0075x; 1.0075x over previous
import functools

import jax
import jax.numpy as jnp
from jax import lax
from jax.experimental import pallas as pl
from jax.experimental.pallas import tpu as pltpu
from jax.experimental.pallas import tpu_sc as plsc

_VOCAB = 1000000
_D = 64
_B = 16384
_L = 200
_N = _B * _L

_VLB = 32768
_NVB = -(-_VOCAB // _VLB)

_NC = 2
_NS = 16
_NW = _NC * _NS
_PER_W = _N // _NW
_CH = 12800
_NCHUNK = _PER_W // _CH

_PL = 50
_NPB = _L // _PL


def _project_body(embt_ref, w_ref, out_ref):
    xt = embt_ref[...]
    w = w_ref[...]
    y = lax.dot_general(w, xt, (((1,), (0,)), ((), ())),
                        preferred_element_type=jnp.float32)
    out_ref[...] = y[0]


def _project_table(embt, fc_w):
    return pl.pallas_call(
        _project_body,
        grid=(_NVB,),
        in_specs=[
            pl.BlockSpec((_D, _VLB), lambda i: (0, i)),
            pl.BlockSpec((1, _D), lambda i: (0, 0)),
        ],
        out_specs=pl.BlockSpec((_VLB,), lambda i: (i,)),
        out_shape=jax.ShapeDtypeStruct((_VOCAB,), jnp.float32),
    )(embt, fc_w)


def _sc_gather(p, idx_flat):
    mesh = plsc.VectorSubcoreMesh(core_axis_name="c", subcore_axis_name="s")

    @functools.partial(
        pl.kernel,
        out_type=jax.ShapeDtypeStruct((_N,), jnp.float32),
        mesh=mesh,
        scratch_types=[
            pltpu.VMEM((_CH,), jnp.int32),
            pltpu.VMEM((_CH,), jnp.int32),
            pltpu.VMEM((_CH,), jnp.int32),
            pltpu.VMEM((_CH,), jnp.float32),
            pltpu.VMEM((_CH,), jnp.float32),
            pltpu.SemaphoreType.DMA,
            pltpu.SemaphoreType.DMA,
            pltpu.SemaphoreType.DMA,
            pltpu.SemaphoreType.DMA,
            pltpu.SemaphoreType.DMA,
        ],
    )
    def gather_kernel(p_hbm, idx_hbm, out_hbm, idx0, idx1, idx2,
                      val0, val1, sem_i, sem_g0, sem_g1, sem_o0, sem_o1):
        wid = lax.axis_index("s") * _NC + lax.axis_index("c")
        base = wid * _PER_W
        ibufs = (idx0, idx1, idx2)
        vbufs = (val0, val1)
        gsems = (sem_g0, sem_g1)
        osems = (sem_o0, sem_o1)

        h_idx = [None] * _NCHUNK
        h_g = [None] * _NCHUNK
        h_out = [None] * _NCHUNK
        for k in range(min(2, _NCHUNK)):
            h_idx[k] = pltpu.async_copy(
                idx_hbm.at[pl.ds(base + k * _CH, _CH)], ibufs[k % 3], sem_i)
        for k in range(_NCHUNK):
            h_idx[k].wait()
            if k >= 2:
                h_out[k - 2].wait()
            h_g[k] = pltpu.async_copy(p_hbm.at[ibufs[k % 3]],
                                      vbufs[k % 2], gsems[k % 2])
            if k == 0 and _NCHUNK > 2:
                h_idx[2] = pltpu.async_copy(
                    idx_hbm.at[pl.ds(base + 2 * _CH, _CH)], ibufs[2], sem_i)
            if k >= 1:
                h_g[k - 1].wait()
                h_out[k - 1] = pltpu.async_copy(
                    vbufs[(k - 1) % 2],
                    out_hbm.at[pl.ds(base + (k - 1) * _CH, _CH)],
                    osems[(k - 1) % 2])
            if k >= 1 and k + 2 < _NCHUNK:
                off = base + (k + 2) * _CH
                h_idx[k + 2] = pltpu.async_copy(
                    idx_hbm.at[pl.ds(off, _CH)], ibufs[(k + 2) % 3], sem_i)
        h_g[_NCHUNK - 1].wait()
        h_out[_NCHUNK - 1] = pltpu.async_copy(
            vbufs[(_NCHUNK - 1) % 2],
            out_hbm.at[pl.ds(base + (_NCHUNK - 1) * _CH, _CH)],
            osems[(_NCHUNK - 1) % 2])
        for k in range(max(0, _NCHUNK - 2), _NCHUNK):
            h_out[k].wait()

    return gather_kernel(p, idx_flat)


def _pool_body(vals_ref, mask_ref, b_ref, out_ref, acc_s, acc_m):
    i = pl.program_id(0)

    @pl.when(i == 0)
    def _init():
        acc_s[...] = jnp.zeros((_B,), jnp.float32)
        acc_m[...] = jnp.zeros((_B,), jnp.float32)

    v = vals_ref[...]
    m = mask_ref[...].astype(jnp.float32)
    s_part = acc_s[...]
    m_part = acc_m[...]
    for s in range(_PL):
        sl = slice(s * _B, (s + 1) * _B)
        s_part = s_part + v[sl] * m[sl]
        m_part = m_part + m[sl]
    acc_s[...] = s_part
    acc_m[...] = m_part

    @pl.when(i == _NPB - 1)
    def _fin():
        out_ref[...] = s_part / m_part + b_ref[0, 0]


def _pool(vals, mask_flat, fc_b):
    b2d = fc_b.reshape(1, 1)
    return pl.pallas_call(
        _pool_body,
        grid=(_NPB,),
        in_specs=[
            pl.BlockSpec((_PL * _B,), lambda i: (i,)),
            pl.BlockSpec((_PL * _B,), lambda i: (i,)),
            pl.BlockSpec((1, 1), lambda i: (0, 0)),
        ],
        out_specs=pl.BlockSpec((_B,), lambda i: (0,)),
        out_shape=jax.ShapeDtypeStruct((_B,), jnp.float32),
        scratch_shapes=[
            pltpu.VMEM((_B,), jnp.float32),
            pltpu.VMEM((_B,), jnp.float32),
        ],
    )(vals, mask_flat, b2d)


def kernel(input_ids, attention_mask, emb_table, fc_w, fc_b):
    p = _project_table(emb_table.T, fc_w)
    idx_flat = input_ids.astype(jnp.int32).T.reshape(_N)
    mask_flat = attention_mask.astype(jnp.int32).T.reshape(_N)
    vals = _sc_gather(p, idx_flat)
    logits = _pool(vals, mask_flat, fc_b)
    return logits.reshape(_B, 1)
